# baseline (speedup 1.0000x reference)
import jax
import jax.numpy as jnp
from jax.experimental import pallas as pl

_NUM_EXPERTS = 64
_TOP_K = 8
_HIDDEN = 4096
_ROWS_PER_BLOCK = 512


def _router_kernel(hs_ref, w_ref, probs_ref, scores_ref, idx_ref):
    hs = hs_ref[...]
    w = w_ref[...]
    logits = jax.lax.dot_general(
        hs, w, (((1,), (1,)), ((), ())), preferred_element_type=jnp.float32
    )
    m = jnp.max(logits, axis=-1, keepdims=True)
    e = jnp.exp(logits - m)
    probs = e / jnp.sum(e, axis=-1, keepdims=True)
    probs_ref[...] = probs

    col = jax.lax.broadcasted_iota(jnp.int32, probs.shape, 1)
    rcol = (jnp.int32(_NUM_EXPERTS - 1) - col).astype(jnp.float32)
    cur = probs
    vals = []
    ridx = []
    for _ in range(_TOP_K):
        mv = jnp.max(cur, axis=-1, keepdims=True)
        rk = jnp.where(cur == mv, rcol, jnp.float32(-1.0))
        rmax = jnp.max(rk, axis=-1, keepdims=True)
        vals.append(mv)
        ridx.append(rmax)
        cur = jnp.where(rk == rmax, jnp.float32(-1.0), cur)
    v = jnp.concatenate(vals, axis=-1)
    i = jnp.int32(_NUM_EXPERTS - 1) - jnp.concatenate(ridx, axis=-1).astype(
        jnp.int32
    )
    v = v / jnp.sum(v, axis=-1, keepdims=True)
    scores_ref[...] = v
    idx_ref[...] = i


def kernel(hidden_states, weight):
    hs = hidden_states.reshape(-1, _HIDDEN)
    n = hs.shape[0]
    grid = n // _ROWS_PER_BLOCK
    probs, scores, idx = pl.pallas_call(
        _router_kernel,
        grid=(grid,),
        in_specs=[
            pl.BlockSpec((_ROWS_PER_BLOCK, _HIDDEN), lambda i: (i, 0)),
            pl.BlockSpec((_NUM_EXPERTS, _HIDDEN), lambda i: (0, 0)),
        ],
        out_specs=[
            pl.BlockSpec((_ROWS_PER_BLOCK, _NUM_EXPERTS), lambda i: (i, 0)),
            pl.BlockSpec((_ROWS_PER_BLOCK, _TOP_K), lambda i: (i, 0)),
            pl.BlockSpec((_ROWS_PER_BLOCK, _TOP_K), lambda i: (i, 0)),
        ],
        out_shape=[
            jax.ShapeDtypeStruct((n, _NUM_EXPERTS), jnp.float32),
            jax.ShapeDtypeStruct((n, _TOP_K), jnp.float32),
            jax.ShapeDtypeStruct((n, _TOP_K), jnp.int32),
        ],
    )(hs, weight)
    return (probs, scores, idx)


# speedup vs baseline: 1.1622x; 1.1622x over previous
import jax
import jax.numpy as jnp
from jax.experimental import pallas as pl

_NUM_EXPERTS = 64
_TOP_K = 8
_HIDDEN = 4096
_ROWS_PER_BLOCK = 1024
_CHUNK = 128


def _router_kernel(hs_ref, w_ref, probs_ref, scores_ref, idx_ref):
    w = w_ref[...]
    col = jax.lax.broadcasted_iota(jnp.int32, (_CHUNK, _NUM_EXPERTS), 1)
    rcol = (jnp.int32(_NUM_EXPERTS - 1) - col).astype(jnp.float32)
    for c in range(_ROWS_PER_BLOCK // _CHUNK):
        sl = pl.ds(c * _CHUNK, _CHUNK)
        hs = hs_ref[sl, :]
        logits = jax.lax.dot_general(
            hs, w, (((1,), (1,)), ((), ())), preferred_element_type=jnp.float32
        )
        m = jnp.max(logits, axis=-1, keepdims=True)
        e = jnp.exp(logits - m)
        probs = e / jnp.sum(e, axis=-1, keepdims=True)
        probs_ref[sl, :] = probs

        cur = probs
        vals = []
        idxs = []
        for _ in range(_TOP_K):
            mv = jnp.max(cur, axis=-1, keepdims=True)
            im = jnp.argmax(cur, axis=-1, keepdims=True).astype(jnp.int32)
            vals.append(mv)
            idxs.append(im)
            cur = jnp.where(col == im, jnp.float32(-1.0), cur)
        v = jnp.concatenate(vals, axis=-1)
        i = jnp.concatenate(idxs, axis=-1)
        v = v / jnp.sum(v, axis=-1, keepdims=True)
        scores_ref[sl, :] = v
        idx_ref[sl, :] = i


def kernel(hidden_states, weight):
    hs = hidden_states.reshape(-1, _HIDDEN)
    n = hs.shape[0]
    grid = n // _ROWS_PER_BLOCK
    probs, scores, idx = pl.pallas_call(
        _router_kernel,
        grid=(grid,),
        in_specs=[
            pl.BlockSpec((_ROWS_PER_BLOCK, _HIDDEN), lambda i: (i, 0)),
            pl.BlockSpec((_NUM_EXPERTS, _HIDDEN), lambda i: (0, 0)),
        ],
        out_specs=[
            pl.BlockSpec((_ROWS_PER_BLOCK, _NUM_EXPERTS), lambda i: (i, 0)),
            pl.BlockSpec((_ROWS_PER_BLOCK, _TOP_K), lambda i: (i, 0)),
            pl.BlockSpec((_ROWS_PER_BLOCK, _TOP_K), lambda i: (i, 0)),
        ],
        out_shape=[
            jax.ShapeDtypeStruct((n, _NUM_EXPERTS), jnp.float32),
            jax.ShapeDtypeStruct((n, _TOP_K), jnp.float32),
            jax.ShapeDtypeStruct((n, _TOP_K), jnp.int32),
        ],
    )(hs, weight)
    return (probs, scores, idx)
